# baseline (speedup 1.0000x reference)
import functools

import jax
import jax.numpy as jnp
from jax import lax
from jax.experimental import pallas as pl
from jax.experimental.pallas import tpu as pltpu
from jax.experimental.pallas import tpu_sc as plsc

VOCAB = 1_000_000
DIM = 64
BATCH = 4096
SEQLEN = 200
TOTAL = BATCH * SEQLEN

_INFO = plsc.get_sparse_core_info()
NC = _INFO.num_cores
NS = _INFO.num_subcores
NW = NC * NS
B_PER_W = TOTAL // NW
CHUNK = 128
N_CHUNKS = B_PER_W // CHUNK


def _body(seq_hbm, table_hbm, out_hbm, idx_v, rows_v, gsem):
    wid = lax.axis_index("s") * NC + lax.axis_index("c")
    base = wid * B_PER_W
    pltpu.sync_copy(seq_hbm.at[wid], idx_v)

    def step(j, _):
        pltpu.async_copy(table_hbm.at[idx_v.at[j]], rows_v, gsem).wait()
        pltpu.sync_copy(rows_v, out_hbm.at[pl.ds(base + j * CHUNK, CHUNK)])
        return 0

    lax.fori_loop(0, N_CHUNKS, step, 0)


@jax.jit
def _lookup(seq_flat, table):
    idx3d = seq_flat.reshape(NW, N_CHUNKS, CHUNK)
    kern = pl.kernel(
        _body,
        out_type=jax.ShapeDtypeStruct((TOTAL, DIM), jnp.float32),
        mesh=plsc.VectorSubcoreMesh(core_axis_name="c", subcore_axis_name="s"),
        scratch_types=[
            pltpu.VMEM((N_CHUNKS, CHUNK), jnp.int32),
            pltpu.VMEM((CHUNK, DIM), jnp.float32),
            pltpu.SemaphoreType.DMA,
        ],
        compiler_params=pltpu.CompilerParams(use_tc_tiling_on_sc=False),
    )
    return kern(idx3d, table)


def kernel(seq, table):
    out = _lookup(seq.reshape(TOTAL).astype(jnp.int32), table)
    return out.reshape(BATCH, SEQLEN, DIM)


# speedup vs baseline: 1.1150x; 1.1150x over previous
import functools

import jax
import jax.numpy as jnp
from jax import lax
from jax.experimental import pallas as pl
from jax.experimental.pallas import tpu as pltpu
from jax.experimental.pallas import tpu_sc as plsc

VOCAB = 1_000_000
DIM = 64
BATCH = 4096
SEQLEN = 200
TOTAL = BATCH * SEQLEN

_INFO = plsc.get_sparse_core_info()
NC = _INFO.num_cores
NS = _INFO.num_subcores
NW = NC * NS
B_PER_W = TOTAL // NW
CHUNK = 128
N_CHUNKS = B_PER_W // CHUNK
NBUF = 8
N_OUTER = N_CHUNKS // NBUF


def _body(seq_hbm, table_hbm, out_hbm, idx_v, *rest):
    bufs = rest[:NBUF]
    gsems = rest[NBUF:2 * NBUF]
    osems = rest[2 * NBUF:3 * NBUF]
    wid = lax.axis_index("s") * NC + lax.axis_index("c")
    base = wid * B_PER_W
    pltpu.sync_copy(seq_hbm.at[wid], idx_v)

    def gather(j, b):
        return pltpu.make_async_copy(
            table_hbm.at[idx_v.at[j]], bufs[b], gsems[b])

    def out_copy(j, b):
        return pltpu.make_async_copy(
            bufs[b], out_hbm.at[pl.ds(base + j * CHUNK, CHUNK)], osems[b])

    for b in range(NBUF):
        gather(b, b).start()

    def step(g, _):
        for b in range(NBUF):
            j = g * NBUF + b
            gather(j, b).wait()
            out_copy(j, b).start()
        for b in range(NBUF):
            j = g * NBUF + b
            out_copy(j, b).wait()

            @pl.when(g + 1 < N_OUTER)
            def _():
                gather(j + NBUF, b).start()
        return 0

    lax.fori_loop(0, N_OUTER, step, 0)


@jax.jit
def _lookup(seq_flat, table):
    idx3d = seq_flat.reshape(NW, N_CHUNKS, CHUNK)
    kern = pl.kernel(
        _body,
        out_type=jax.ShapeDtypeStruct((TOTAL, DIM), jnp.float32),
        mesh=plsc.VectorSubcoreMesh(core_axis_name="c", subcore_axis_name="s"),
        scratch_types=(
            [pltpu.VMEM((N_CHUNKS, CHUNK), jnp.int32)]
            + [pltpu.VMEM((CHUNK, DIM), jnp.float32) for _ in range(NBUF)]
            + [pltpu.SemaphoreType.DMA for _ in range(2 * NBUF)]
        ),
        compiler_params=pltpu.CompilerParams(use_tc_tiling_on_sc=False),
    )
    return kern(idx3d, table)


def kernel(seq, table):
    out = _lookup(seq.reshape(TOTAL).astype(jnp.int32), table)
    return out.reshape(BATCH, SEQLEN, DIM)


# speedup vs baseline: 1.4852x; 1.3321x over previous
import functools

import jax
import jax.numpy as jnp
from jax import lax
from jax.experimental import pallas as pl
from jax.experimental.pallas import tpu as pltpu
from jax.experimental.pallas import tpu_sc as plsc

VOCAB = 1_000_000
DIM = 64
PAD = 128
BATCH = 4096
SEQLEN = 200
TOTAL = BATCH * SEQLEN

_INFO = plsc.get_sparse_core_info()
NC = _INFO.num_cores
NS = _INFO.num_subcores
NW = NC * NS
B_PER_W = TOTAL // NW
CHUNK = 128
N_CHUNKS = B_PER_W // CHUNK
NBUF = 8
N_OUTER = N_CHUNKS // NBUF


def _body(seq_hbm, table_hbm, out_hbm, idx_v, *rest):
    bufs = rest[:NBUF]
    gsems = rest[NBUF:2 * NBUF]
    osems = rest[2 * NBUF:3 * NBUF]
    wid = lax.axis_index("s") * NC + lax.axis_index("c")
    base = wid * B_PER_W
    pltpu.sync_copy(seq_hbm.at[wid], idx_v)

    def gather(j, b):
        return pltpu.make_async_copy(
            table_hbm.at[idx_v.at[j]], bufs[b], gsems[b])

    def out_copy(j, b):
        return pltpu.make_async_copy(
            bufs[b],
            out_hbm.at[pl.ds(base + j * CHUNK, CHUNK), pl.ds(0, DIM)],
            osems[b])

    for b in range(NBUF):
        gather(b, b).start()

    def step(g, _):
        for b in range(NBUF):
            j = g * NBUF + b
            gather(j, b).wait()
            out_copy(j, b).start()
        for b in range(NBUF):
            j = g * NBUF + b
            out_copy(j, b).wait()

            @pl.when(g + 1 < N_OUTER)
            def _():
                gather(j + NBUF, b).start()
        return 0

    lax.fori_loop(0, N_OUTER, step, 0)


@jax.jit
def _lookup(seq, table):
    idx3d = seq.reshape(NW, N_CHUNKS, CHUNK)
    kern = pl.kernel(
        _body,
        out_type=jax.ShapeDtypeStruct((TOTAL, PAD), jnp.float32),
        mesh=plsc.VectorSubcoreMesh(core_axis_name="c", subcore_axis_name="s"),
        scratch_types=(
            [pltpu.VMEM((N_CHUNKS, CHUNK), jnp.int32)]
            + [pltpu.VMEM((CHUNK, DIM), jnp.float32) for _ in range(NBUF)]
            + [pltpu.SemaphoreType.DMA for _ in range(2 * NBUF)]
        ),
        compiler_params=pltpu.CompilerParams(use_tc_tiling_on_sc=False),
    )
    return kern(idx3d, table)[:, :DIM]


def kernel(seq, table):
    out = _lookup(seq.reshape(TOTAL).astype(jnp.int32), table)
    return out.reshape(BATCH, SEQLEN, DIM)
